# baseline (speedup 1.0000x reference)
import functools

import jax
import jax.numpy as jnp
from jax import lax
from jax.experimental import pallas as pl
from jax.experimental.pallas import tpu as pltpu
from jax.experimental.pallas import tpu_sc as plsc

N = 10000
NPAD = 10240
E = 160000
F = 128
BN_ROWS = 1000
NT = 16
EPAD = 163840
EPT = EPAD // NT
B = 128
NB = EPT // B
WH = 8
ACC = 10112
EPS = 1e-5


def _mm_first_body(x_ref, w_ref, o_ref):
    o_ref[0] = jnp.dot(x_ref[...], w_ref[...],
                       preferred_element_type=jnp.float32)


def _mm_first(x, w, c_out):
    din = x.shape[1]
    grid = (N // BN_ROWS, c_out)
    return pl.pallas_call(
        _mm_first_body,
        grid=grid,
        in_specs=[
            pl.BlockSpec((BN_ROWS, din), lambda i, q: (i, 0)),
            pl.BlockSpec((din, F), lambda i, q: (0, q)),
        ],
        out_specs=pl.BlockSpec((1, BN_ROWS, F), lambda i, q: (q, i, 0)),
        out_shape=jax.ShapeDtypeStruct((c_out, NPAD, F), jnp.float32),
    )(x, w)


def _mm_bn_body(c_in, x_ref, w_ref, ssum_ref, ssq_ref, b_ref, g_ref,
                be_ref, o_ref):
    acc = jnp.zeros((BN_ROWS, F), dtype=jnp.float32)
    for qi in range(c_in):
        m = ssum_ref[qi].sum(axis=0) * (1.0 / N)
        v = ssq_ref[qi].sum(axis=0) * (1.0 / N) - m * m
        s = g_ref[qi] * lax.rsqrt(v + EPS)
        c = be_ref[qi] - m * s
        h = jnp.maximum(x_ref[qi] + b_ref[qi], 0.0) * s + c
        acc += jnp.dot(h, w_ref[qi], preferred_element_type=jnp.float32)
    o_ref[0] = acc


def _mm_bn(agg, w, ssum, ssq, b, g, be, c_in, c_out):
    wv = w.reshape(c_in, F, c_out * F)
    grid = (N // BN_ROWS, c_out)
    return pl.pallas_call(
        functools.partial(_mm_bn_body, c_in),
        grid=grid,
        in_specs=[
            pl.BlockSpec((c_in, BN_ROWS, F), lambda i, q: (0, i, 0)),
            pl.BlockSpec((c_in, F, F), lambda i, q: (0, 0, q)),
            pl.BlockSpec((c_in, 8, F), lambda i, q: (0, 0, 0)),
            pl.BlockSpec((c_in, 8, F), lambda i, q: (0, 0, 0)),
            pl.BlockSpec((c_in, F), lambda i, q: (0, 0)),
            pl.BlockSpec((c_in, F), lambda i, q: (0, 0)),
            pl.BlockSpec((c_in, F), lambda i, q: (0, 0)),
        ],
        out_specs=pl.BlockSpec((1, BN_ROWS, F), lambda i, q: (q, i, 0)),
        out_shape=jax.ShapeDtypeStruct((c_out, NPAD, F), jnp.float32),
    )(agg, wv, ssum, ssq, b, g, be)


def _stats_body(x_ref, b_ref, ssum_ref, ssq_ref):
    i = pl.program_id(0)

    @pl.when(i == 0)
    def _init():
        ssum_ref[...] = jnp.zeros_like(ssum_ref)
        ssq_ref[...] = jnp.zeros_like(ssq_ref)

    c_in = x_ref.shape[0]
    for qi in range(c_in):
        r = jnp.maximum(x_ref[qi] + b_ref[qi], 0.0)
        ssum_ref[qi] += r.reshape(BN_ROWS // 8, 8, F).sum(axis=0)
        ssq_ref[qi] += (r * r).reshape(BN_ROWS // 8, 8, F).sum(axis=0)


def _stats(agg, b, c_in):
    grid = (N // BN_ROWS,)
    return pl.pallas_call(
        _stats_body,
        grid=grid,
        in_specs=[
            pl.BlockSpec((c_in, BN_ROWS, F), lambda i: (0, i, 0)),
            pl.BlockSpec((c_in, F), lambda i: (0, 0)),
        ],
        out_specs=[
            pl.BlockSpec((c_in, 8, F), lambda i: (0, 0, 0)),
            pl.BlockSpec((c_in, 8, F), lambda i: (0, 0, 0)),
        ],
        out_shape=[
            jax.ShapeDtypeStruct((c_in, 8, F), jnp.float32),
            jax.ShapeDtypeStruct((c_in, 8, F), jnp.float32),
        ],
    )(agg, b)


def _final_body(x_ref, ssum_ref, ssq_ref, b_ref, g_ref, be_ref, o_ref):
    c_in = x_ref.shape[0]
    outs = []
    for qi in range(c_in):
        m = ssum_ref[qi].sum(axis=0) * (1.0 / N)
        v = ssq_ref[qi].sum(axis=0) * (1.0 / N) - m * m
        s = g_ref[qi] * lax.rsqrt(v + EPS)
        c = be_ref[qi] - m * s
        outs.append(jnp.maximum(x_ref[qi] + b_ref[qi], 0.0) * s + c)
    o_ref[...] = jnp.concatenate(outs, axis=-1)


def _final(agg, ssum, ssq, b, g, be, c_in):
    grid = (N // BN_ROWS,)
    return pl.pallas_call(
        _final_body,
        grid=grid,
        in_specs=[
            pl.BlockSpec((c_in, BN_ROWS, F), lambda i: (0, i, 0)),
            pl.BlockSpec((c_in, 8, F), lambda i: (0, 0, 0)),
            pl.BlockSpec((c_in, 8, F), lambda i: (0, 0, 0)),
            pl.BlockSpec((c_in, F), lambda i: (0, 0)),
            pl.BlockSpec((c_in, F), lambda i: (0, 0)),
            pl.BlockSpec((c_in, F), lambda i: (0, 0)),
        ],
        out_specs=pl.BlockSpec((BN_ROWS, c_in * F), lambda i: (i, 0)),
        out_shape=jax.ShapeDtypeStruct((N, c_in * F), jnp.float32),
    )(agg, ssum, ssq, b, g, be)


def _sc_scatter_kernel(c_out, t_ref, ei_ref, out_ref,
                       idxw, dstw, buf_a, buf_b, spacc,
                       gsem_a, gsem_b, rsem0, rsem1):
    c = lax.axis_index("c")
    s = lax.axis_index("s")
    rows = ACC // NT

    zero16 = jnp.zeros((16,), jnp.float32)
    NWIN = NB // (2 * WH)

    def _refill(P, wstart, sem, sync):
        if sync:
            pltpu.sync_copy(ei_ref.at[0, s, pl.ds(wstart, WH)],
                            idxw.at[pl.ds(P * WH, WH)])
            pltpu.sync_copy(ei_ref.at[1, s, pl.ds(wstart, WH)],
                            dstw.at[pl.ds(P * WH, WH)])
        else:
            pltpu.async_copy(ei_ref.at[0, s, pl.ds(wstart, WH)],
                             idxw.at[pl.ds(P * WH, WH)], sem)
            pltpu.async_copy(ei_ref.at[1, s, pl.ds(wstart, WH)],
                             dstw.at[pl.ds(P * WH, WH)], sem)

    def _wait_refill(P, sem):
        pltpu.make_async_copy(ei_ref.at[0, s, pl.ds(0, WH)],
                              idxw.at[pl.ds(P * WH, WH)], sem).wait()
        pltpu.make_async_copy(ei_ref.at[1, s, pl.ds(0, WH)],
                              dstw.at[pl.ds(P * WH, WH)], sem).wait()

    for qi in range(c_out // 2):
        q = qi * 2 + c
        off = jnp.zeros((16,), jnp.int32) + q * NPAD

        def _add_off(P, _ignored=None):
            def _row(i, _):
                def _col(j, _):
                    idxw[P * WH + i, pl.ds(j * 16, 16)] = (
                        idxw[P * WH + i, pl.ds(j * 16, 16)] + off)
                    return 0
                return lax.fori_loop(0, B // 16, _col, 0)
            lax.fori_loop(0, WH, _row, 0)

        def _zrow(i, _):
            def _zcol(j, _):
                buf_a[i, pl.ds(j * 16, 16)] = zero16
                return 0
            return lax.fori_loop(0, F // 16, _zcol, 0)

        lax.fori_loop(0, B, _zrow, 0)
        for k in range(rows // B):
            pltpu.sync_copy(buf_a, spacc.at[pl.ds(s * rows + k * B, B)])
        pltpu.sync_copy(buf_a.at[pl.ds(0, rows % B)],
                        spacc.at[pl.ds(s * rows + (rows // B) * B, rows % B)])
        plsc.subcore_barrier()

        _refill(0, 0, rsem0, True)
        _add_off(0)
        _refill(1, WH, rsem1, False)
        pltpu.async_copy(t_ref.at[idxw.at[0]], buf_a, gsem_a)
        pltpu.async_copy(t_ref.at[idxw.at[1]], buf_b, gsem_b)

        def _ring(w, _):
            for P in (0, 1):
                base = (2 * w + P) * WH
                other = 1 - P
                osem = rsem1 if P == 0 else rsem0

                if P == 0:
                    _wait_refill(other, osem)
                    _add_off(other)
                else:
                    @pl.when(w < NWIN - 1)
                    def _stage_other():
                        _wait_refill(other, osem)
                        _add_off(other)

                def _pair(k, _):
                    j0 = base + 2 * k
                    r0 = P * WH + 2 * k
                    pltpu.make_async_copy(t_ref.at[idxw.at[0]], buf_a,
                                          gsem_a).wait()
                    pltpu.sync_copy(buf_a, spacc.at[dstw.at[r0]], add=True)

                    @pl.when(j0 + 2 < NB)
                    def _refire_a():
                        r2 = (r0 + 2) % (2 * WH)
                        pltpu.async_copy(t_ref.at[idxw.at[r2]], buf_a,
                                         gsem_a)

                    pltpu.make_async_copy(t_ref.at[idxw.at[0]], buf_b,
                                          gsem_b).wait()
                    pltpu.sync_copy(buf_b, spacc.at[dstw.at[r0 + 1]],
                                    add=True)

                    @pl.when(j0 + 3 < NB)
                    def _refire_b():
                        r3 = (r0 + 3) % (2 * WH)
                        pltpu.async_copy(t_ref.at[idxw.at[r3]], buf_b,
                                         gsem_b)
                    return 0

                lax.fori_loop(0, WH // 2, _pair, 0)

                @pl.when(w < NWIN - 1)
                def _refill_self():
                    nstart = base + 2 * WH
                    sem = rsem0 if P == 0 else rsem1
                    _refill(P, nstart, sem, False)
            return 0

        lax.fori_loop(0, NWIN, _ring, 0)
        plsc.subcore_barrier()

        pltpu.sync_copy(spacc.at[pl.ds(s * rows, rows)],
                        out_ref.at[pl.ds(q * NPAD + s * rows, rows)])
        plsc.subcore_barrier()


def _sc_scatter(t, ei, c_out):
    mesh = plsc.VectorSubcoreMesh(core_axis_name="c", subcore_axis_name="s")
    kfn = pl.kernel(
        functools.partial(_sc_scatter_kernel, c_out),
        mesh=mesh,
        out_type=jax.ShapeDtypeStruct((c_out * NPAD, F), jnp.float32),
        scratch_types=[
            pltpu.VMEM((2 * WH, B), jnp.int32),
            pltpu.VMEM((2 * WH, B), jnp.int32),
            pltpu.VMEM((B, F), jnp.float32),
            pltpu.VMEM((B, F), jnp.float32),
            pltpu.VMEM_SHARED((ACC, F), jnp.float32),
            pltpu.SemaphoreType.DMA,
            pltpu.SemaphoreType.DMA,
            pltpu.SemaphoreType.DMA,
            pltpu.SemaphoreType.DMA,
        ],
    )
    out = kfn(t.reshape(c_out * NPAD, F), ei)
    return out.reshape(c_out, NPAD, F)


def kernel(x, edge_index, W0, b0, g0, be0, W1, b1, g1, be1, W2, b2, g2, be2,
           W3, b3, g3, be3):
    ei32 = edge_index.astype(jnp.int32)
    npad_e = EPAD - E
    pad_src = (jnp.arange(npad_e, dtype=jnp.int32) * 37) % N
    pad_dst = N + (jnp.arange(npad_e, dtype=jnp.int32) % (ACC - N))
    ei = jnp.concatenate(
        [ei32, jnp.stack([pad_src, pad_dst])], axis=1).reshape(2, NT, NB, B)
    params = [(W0, b0, g0, be0), (W1, b1, g1, be1), (W2, b2, g2, be2),
              (W3, b3, g3, be3)]

    c_out = W0.shape[1] // F
    t = _mm_first(x, W0, c_out)
    agg = _sc_scatter(t, ei, c_out)
    bq = b0.reshape(c_out, F)
    ssum, ssq = _stats(agg, bq, c_out)

    for li in range(1, 4):
        W, b, g, be = params[li]
        _, bp, gp, bep = params[li - 1]
        c_in = agg.shape[0]
        c_out = W.shape[1] // F
        t = _mm_bn(agg, W, ssum, ssq, bp.reshape(c_in, F),
                   gp.reshape(c_in, F), bep.reshape(c_in, F), c_in, c_out)
        agg = _sc_scatter(t, ei, c_out)
        ssum, ssq = _stats(agg, b.reshape(c_out, F), c_out)

    c_in = agg.shape[0]
    return _final(agg, ssum, ssq, b3.reshape(c_in, F), g3.reshape(c_in, F),
                  be3.reshape(c_in, F), c_in)


# speedup vs baseline: 1.0347x; 1.0347x over previous
import functools

import jax
import jax.numpy as jnp
from jax import lax
from jax.experimental import pallas as pl
from jax.experimental.pallas import tpu as pltpu
from jax.experimental.pallas import tpu_sc as plsc

N = 10000
NPAD = 10240
E = 160000
F = 128
BN_ROWS = 1000
NT = 16
EPAD = 163840
EPT = EPAD // NT
B = 128
NB = EPT // B
WH = 8
ACC = 10112
EPS = 1e-5


def _mm_first_body(x_ref, w_ref, o_ref):
    o_ref[0] = jnp.dot(x_ref[...], w_ref[...],
                       preferred_element_type=jnp.float32)


def _mm_first(x, w, c_out):
    din = x.shape[1]
    grid = (N // BN_ROWS, c_out)
    return pl.pallas_call(
        _mm_first_body,
        grid=grid,
        in_specs=[
            pl.BlockSpec((BN_ROWS, din), lambda i, q: (i, 0)),
            pl.BlockSpec((din, F), lambda i, q: (0, q)),
        ],
        out_specs=pl.BlockSpec((1, BN_ROWS, F), lambda i, q: (q, i, 0)),
        out_shape=jax.ShapeDtypeStruct((c_out, NPAD, F), jnp.float32),
    )(x, w)


def _mm_bn_body(c_in, x_ref, w_ref, ssum_ref, ssq_ref, b_ref, g_ref,
                be_ref, o_ref):
    acc = jnp.zeros((BN_ROWS, F), dtype=jnp.float32)
    for qi in range(c_in):
        m = ssum_ref[qi].sum(axis=0) * (1.0 / N)
        v = ssq_ref[qi].sum(axis=0) * (1.0 / N) - m * m
        s = g_ref[qi] * lax.rsqrt(v + EPS)
        c = be_ref[qi] - m * s
        h = jnp.maximum(x_ref[qi] + b_ref[qi], 0.0) * s + c
        acc += jnp.dot(h, w_ref[qi], preferred_element_type=jnp.float32)
    o_ref[0] = acc


def _mm_bn(agg, w, ssum, ssq, b, g, be, c_in, c_out):
    wv = w.reshape(c_in, F, c_out * F)
    grid = (N // BN_ROWS, c_out)
    return pl.pallas_call(
        functools.partial(_mm_bn_body, c_in),
        grid=grid,
        in_specs=[
            pl.BlockSpec((c_in, BN_ROWS, F), lambda i, q: (0, i, 0)),
            pl.BlockSpec((c_in, F, F), lambda i, q: (0, 0, q)),
            pl.BlockSpec((c_in, 8, F), lambda i, q: (0, 0, 0)),
            pl.BlockSpec((c_in, 8, F), lambda i, q: (0, 0, 0)),
            pl.BlockSpec((c_in, F), lambda i, q: (0, 0)),
            pl.BlockSpec((c_in, F), lambda i, q: (0, 0)),
            pl.BlockSpec((c_in, F), lambda i, q: (0, 0)),
        ],
        out_specs=pl.BlockSpec((1, BN_ROWS, F), lambda i, q: (q, i, 0)),
        out_shape=jax.ShapeDtypeStruct((c_out, NPAD, F), jnp.float32),
    )(agg, wv, ssum, ssq, b, g, be)


def _mm_bn2_body(x_a, x_b, w_a, w_b, ssum_a, ssq_a, ssum_b, ssq_b,
                 b_a, g_a, be_a, b_b, g_b, be_b, o_ref):
    acc = jnp.zeros((BN_ROWS, F), dtype=jnp.float32)
    for x_ref, w_ref, ssum_ref, ssq_ref, b_ref, g_ref, be_ref in (
            (x_a, w_a, ssum_a, ssq_a, b_a, g_a, be_a),
            (x_b, w_b, ssum_b, ssq_b, b_b, g_b, be_b)):
        for qi in range(2):
            m = ssum_ref[qi].sum(axis=0) * (1.0 / N)
            v = ssq_ref[qi].sum(axis=0) * (1.0 / N) - m * m
            s = g_ref[qi] * lax.rsqrt(v + EPS)
            c = be_ref[qi] - m * s
            h = jnp.maximum(x_ref[qi] + b_ref[qi], 0.0) * s + c
            acc += jnp.dot(h, w_ref[qi], preferred_element_type=jnp.float32)
    o_ref[0] = acc


def _mm_bn2(agg_a, agg_b, w_a, w_b, ss_a, ss_b, p_a, p_b, c_out):
    wva = w_a.reshape(2, F, c_out * F)
    wvb = w_b.reshape(2, F, c_out * F)
    grid = (N // BN_ROWS, c_out)
    sspec = pl.BlockSpec((2, 8, F), lambda i, q: (0, 0, 0))
    pspec = pl.BlockSpec((2, F), lambda i, q: (0, 0))
    return pl.pallas_call(
        _mm_bn2_body,
        grid=grid,
        in_specs=[
            pl.BlockSpec((2, BN_ROWS, F), lambda i, q: (0, i, 0)),
            pl.BlockSpec((2, BN_ROWS, F), lambda i, q: (0, i, 0)),
            pl.BlockSpec((2, F, F), lambda i, q: (0, 0, q)),
            pl.BlockSpec((2, F, F), lambda i, q: (0, 0, q)),
            sspec, sspec, sspec, sspec,
            pspec, pspec, pspec, pspec, pspec, pspec,
        ],
        out_specs=pl.BlockSpec((1, BN_ROWS, F), lambda i, q: (q, i, 0)),
        out_shape=jax.ShapeDtypeStruct((c_out, NPAD, F), jnp.float32),
    )(agg_a, agg_b, wva, wvb, ss_a[0], ss_a[1], ss_b[0], ss_b[1],
      p_a[0], p_a[1], p_a[2], p_b[0], p_b[1], p_b[2])


def _stats_body(x_ref, b_ref, ssum_ref, ssq_ref):
    i = pl.program_id(0)

    @pl.when(i == 0)
    def _init():
        ssum_ref[...] = jnp.zeros_like(ssum_ref)
        ssq_ref[...] = jnp.zeros_like(ssq_ref)

    c_in = x_ref.shape[0]
    for qi in range(c_in):
        r = jnp.maximum(x_ref[qi] + b_ref[qi], 0.0)
        ssum_ref[qi] += r.reshape(BN_ROWS // 8, 8, F).sum(axis=0)
        ssq_ref[qi] += (r * r).reshape(BN_ROWS // 8, 8, F).sum(axis=0)


def _stats(agg, b, c_in):
    grid = (N // BN_ROWS,)
    return pl.pallas_call(
        _stats_body,
        grid=grid,
        in_specs=[
            pl.BlockSpec((c_in, BN_ROWS, F), lambda i: (0, i, 0)),
            pl.BlockSpec((c_in, F), lambda i: (0, 0)),
        ],
        out_specs=[
            pl.BlockSpec((c_in, 8, F), lambda i: (0, 0, 0)),
            pl.BlockSpec((c_in, 8, F), lambda i: (0, 0, 0)),
        ],
        out_shape=[
            jax.ShapeDtypeStruct((c_in, 8, F), jnp.float32),
            jax.ShapeDtypeStruct((c_in, 8, F), jnp.float32),
        ],
    )(agg, b)


def _final_body(x_ref, ssum_ref, ssq_ref, b_ref, g_ref, be_ref, o_ref):
    c_in = x_ref.shape[0]
    outs = []
    for qi in range(c_in):
        m = ssum_ref[qi].sum(axis=0) * (1.0 / N)
        v = ssq_ref[qi].sum(axis=0) * (1.0 / N) - m * m
        s = g_ref[qi] * lax.rsqrt(v + EPS)
        c = be_ref[qi] - m * s
        outs.append(jnp.maximum(x_ref[qi] + b_ref[qi], 0.0) * s + c)
    o_ref[...] = jnp.concatenate(outs, axis=-1)


def _final(agg, ssum, ssq, b, g, be, c_in):
    grid = (N // BN_ROWS,)
    return pl.pallas_call(
        _final_body,
        grid=grid,
        in_specs=[
            pl.BlockSpec((c_in, BN_ROWS, F), lambda i: (0, i, 0)),
            pl.BlockSpec((c_in, 8, F), lambda i: (0, 0, 0)),
            pl.BlockSpec((c_in, 8, F), lambda i: (0, 0, 0)),
            pl.BlockSpec((c_in, F), lambda i: (0, 0)),
            pl.BlockSpec((c_in, F), lambda i: (0, 0)),
            pl.BlockSpec((c_in, F), lambda i: (0, 0)),
        ],
        out_specs=pl.BlockSpec((BN_ROWS, c_in * F), lambda i: (i, 0)),
        out_shape=jax.ShapeDtypeStruct((N, c_in * F), jnp.float32),
    )(agg, ssum, ssq, b, g, be)


def _sc_scatter_kernel(c_out, t_ref, ei_ref, out_ref,
                       idxw, dstw, buf_a, buf_b, spacc,
                       gsem_a, gsem_b, rsem0, rsem1):
    c = lax.axis_index("c")
    s = lax.axis_index("s")
    rows = ACC // NT

    zero16 = jnp.zeros((16,), jnp.float32)
    NWIN = NB // (2 * WH)

    def _refill(P, wstart, sem, sync):
        if sync:
            pltpu.sync_copy(ei_ref.at[0, s, pl.ds(wstart, WH)],
                            idxw.at[pl.ds(P * WH, WH)])
            pltpu.sync_copy(ei_ref.at[1, s, pl.ds(wstart, WH)],
                            dstw.at[pl.ds(P * WH, WH)])
        else:
            pltpu.async_copy(ei_ref.at[0, s, pl.ds(wstart, WH)],
                             idxw.at[pl.ds(P * WH, WH)], sem)
            pltpu.async_copy(ei_ref.at[1, s, pl.ds(wstart, WH)],
                             dstw.at[pl.ds(P * WH, WH)], sem)

    def _wait_refill(P, sem):
        pltpu.make_async_copy(ei_ref.at[0, s, pl.ds(0, WH)],
                              idxw.at[pl.ds(P * WH, WH)], sem).wait()
        pltpu.make_async_copy(ei_ref.at[1, s, pl.ds(0, WH)],
                              dstw.at[pl.ds(P * WH, WH)], sem).wait()

    for qi in range(c_out // 2):
        q = qi * 2 + c
        off = jnp.zeros((16,), jnp.int32) + q * NPAD

        def _add_off(P, _ignored=None):
            def _row(i, _):
                def _col(j, _):
                    idxw[P * WH + i, pl.ds(j * 16, 16)] = (
                        idxw[P * WH + i, pl.ds(j * 16, 16)] + off)
                    return 0
                return lax.fori_loop(0, B // 16, _col, 0)
            lax.fori_loop(0, WH, _row, 0)

        def _zrow(i, _):
            def _zcol(j, _):
                buf_a[i, pl.ds(j * 16, 16)] = zero16
                return 0
            return lax.fori_loop(0, F // 16, _zcol, 0)

        lax.fori_loop(0, B, _zrow, 0)
        for k in range(rows // B):
            pltpu.sync_copy(buf_a, spacc.at[pl.ds(s * rows + k * B, B)])
        pltpu.sync_copy(buf_a.at[pl.ds(0, rows % B)],
                        spacc.at[pl.ds(s * rows + (rows // B) * B, rows % B)])
        plsc.subcore_barrier()

        _refill(0, 0, rsem0, True)
        _add_off(0)
        _refill(1, WH, rsem1, False)
        pltpu.async_copy(t_ref.at[idxw.at[0]], buf_a, gsem_a)
        pltpu.async_copy(t_ref.at[idxw.at[1]], buf_b, gsem_b)

        def _ring(w, _):
            for P in (0, 1):
                base = (2 * w + P) * WH
                other = 1 - P
                osem = rsem1 if P == 0 else rsem0

                if P == 0:
                    _wait_refill(other, osem)
                    _add_off(other)
                else:
                    @pl.when(w < NWIN - 1)
                    def _stage_other():
                        _wait_refill(other, osem)
                        _add_off(other)

                def _pair(k, _):
                    j0 = base + 2 * k
                    r0 = P * WH + 2 * k
                    pltpu.make_async_copy(t_ref.at[idxw.at[0]], buf_a,
                                          gsem_a).wait()
                    pltpu.sync_copy(buf_a, spacc.at[dstw.at[r0]], add=True)

                    @pl.when(j0 + 2 < NB)
                    def _refire_a():
                        r2 = (r0 + 2) % (2 * WH)
                        pltpu.async_copy(t_ref.at[idxw.at[r2]], buf_a,
                                         gsem_a)

                    pltpu.make_async_copy(t_ref.at[idxw.at[0]], buf_b,
                                          gsem_b).wait()
                    pltpu.sync_copy(buf_b, spacc.at[dstw.at[r0 + 1]],
                                    add=True)

                    @pl.when(j0 + 3 < NB)
                    def _refire_b():
                        r3 = (r0 + 3) % (2 * WH)
                        pltpu.async_copy(t_ref.at[idxw.at[r3]], buf_b,
                                         gsem_b)
                    return 0

                lax.fori_loop(0, WH // 2, _pair, 0)

                @pl.when(w < NWIN - 1)
                def _refill_self():
                    nstart = base + 2 * WH
                    sem = rsem0 if P == 0 else rsem1
                    _refill(P, nstart, sem, False)
            return 0

        lax.fori_loop(0, NWIN, _ring, 0)
        plsc.subcore_barrier()

        pltpu.sync_copy(spacc.at[pl.ds(s * rows, rows)],
                        out_ref.at[pl.ds(q * NPAD + s * rows, rows)])
        plsc.subcore_barrier()


def _sc_scatter(t, ei, c_out):
    mesh = plsc.VectorSubcoreMesh(core_axis_name="c", subcore_axis_name="s")
    kfn = pl.kernel(
        functools.partial(_sc_scatter_kernel, c_out),
        mesh=mesh,
        out_type=jax.ShapeDtypeStruct((c_out * NPAD, F), jnp.float32),
        scratch_types=[
            pltpu.VMEM((2 * WH, B), jnp.int32),
            pltpu.VMEM((2 * WH, B), jnp.int32),
            pltpu.VMEM((B, F), jnp.float32),
            pltpu.VMEM((B, F), jnp.float32),
            pltpu.VMEM_SHARED((ACC, F), jnp.float32),
            pltpu.SemaphoreType.DMA,
            pltpu.SemaphoreType.DMA,
            pltpu.SemaphoreType.DMA,
            pltpu.SemaphoreType.DMA,
        ],
    )
    out = kfn(t.reshape(c_out * NPAD, F), ei)
    return out.reshape(c_out, NPAD, F)


def kernel(x, edge_index, W0, b0, g0, be0, W1, b1, g1, be1, W2, b2, g2, be2,
           W3, b3, g3, be3):
    ei32 = edge_index.astype(jnp.int32)
    npad_e = EPAD - E
    pad_src = (jnp.arange(npad_e, dtype=jnp.int32) * 37) % N
    pad_dst = N + (jnp.arange(npad_e, dtype=jnp.int32) % (ACC - N))
    ei = jnp.concatenate(
        [ei32, jnp.stack([pad_src, pad_dst])], axis=1).reshape(2, NT, NB, B)

    def halves(v):
        d = v.shape[0] // 2
        return v[:d].reshape(2, F), v[d:].reshape(2, F)

    t_a = _mm_first(x, W0[:, :256], 2)
    t_b = _mm_first(x, W0[:, 256:], 2)
    agg_a = _sc_scatter(t_a, ei, 2)
    agg_b = _sc_scatter(t_b, ei, 2)
    b_ah, b_bh = halves(b0)
    ss_a = _stats(agg_a, b_ah, 2)
    ss_b = _stats(agg_b, b_bh, 2)

    for li in range(1, 3):
        W, b, g, be = [(W1, b1, g1, be1), (W2, b2, g2, be2)][li - 1]
        _, bp, gp, bep = [(W0, b0, g0, be0), (W1, b1, g1, be1)][li - 1]
        p_a = (halves(bp)[0], halves(gp)[0], halves(bep)[0])
        p_b = (halves(bp)[1], halves(gp)[1], halves(bep)[1])
        t_a2 = _mm_bn2(agg_a, agg_b, W[:256, :256], W[256:, :256],
                       ss_a, ss_b, p_a, p_b, 2)
        t_b2 = _mm_bn2(agg_a, agg_b, W[:256, 256:], W[256:, 256:],
                       ss_a, ss_b, p_a, p_b, 2)
        agg_a = _sc_scatter(t_a2, ei, 2)
        agg_b = _sc_scatter(t_b2, ei, 2)
        ss_a = _stats(agg_a, halves(b)[0], 2)
        ss_b = _stats(agg_b, halves(b)[1], 2)

    p_a = (halves(b2)[0], halves(g2)[0], halves(be2)[0])
    p_b = (halves(b2)[1], halves(g2)[1], halves(be2)[1])
    t3 = _mm_bn2(agg_a, agg_b, W3[:256], W3[256:], ss_a, ss_b, p_a, p_b, 2)
    agg3 = _sc_scatter(t3, ei, 2)
    b3h = b3.reshape(2, F)
    ssum3, ssq3 = _stats(agg3, b3h, 2)
    return _final(agg3, ssum3, ssq3, b3h, g3.reshape(2, F),
                  be3.reshape(2, F), 2)
